# baseline (speedup 1.0000x reference)
import functools

import jax
import jax.numpy as jnp
from jax import lax
from jax.experimental import pallas as pl
from jax.experimental.pallas import tpu as pltpu
from jax.experimental.pallas import tpu_sc as plsc

NC = 2
NS = 16
NW = NC * NS
L = 128
NBUF = 7


def kernel(X, W):
    B, H = X.shape
    V, D = W.shape
    bpw = B // NW
    assert bpw * NW == B and bpw == L
    ngrp = H // NBUF
    tail = H - ngrp * NBUF

    Xt = X.T.astype(jnp.int32)

    mesh = plsc.VectorSubcoreMesh(core_axis_name="c", subcore_axis_name="s")

    @functools.partial(
        pl.kernel,
        out_type=jax.ShapeDtypeStruct((H, B, D), jnp.float32),
        mesh=mesh,
        scratch_types=[
            pltpu.VMEM((H, L), jnp.int32),
            [pltpu.VMEM((L, D), jnp.float32) for _ in range(NBUF)],
            [pltpu.SemaphoreType.DMA for _ in range(NBUF)],
            [pltpu.SemaphoreType.DMA for _ in range(NBUF)],
        ],
    )
    def emb(x_hbm, w_hbm, out_hbm, idx_v, bufs, gsems, osems):
        wid = lax.axis_index("s") * NC + lax.axis_index("c")
        b0 = wid * L
        pltpu.sync_copy(
            x_hbm.at[pl.ds(0, 8), pl.ds(b0, L)], idx_v.at[pl.ds(0, 8)]
        )

        def start_gather(b, h):
            pltpu.make_async_copy(
                w_hbm.at[idx_v.at[h]], bufs[b], gsems[b]
            ).start()

        def wait_gather(b, h):
            pltpu.make_async_copy(
                w_hbm.at[idx_v.at[h]], bufs[b], gsems[b]
            ).wait()

        def start_out(b, h):
            pltpu.make_async_copy(
                bufs[b], out_hbm.at[h, pl.ds(b0, L)], osems[b]
            ).start()

        def wait_out(b, h):
            pltpu.make_async_copy(
                bufs[b], out_hbm.at[h, pl.ds(b0, L)], osems[b]
            ).wait()

        for b in range(NBUF):
            start_gather(b, b)

        pltpu.sync_copy(
            x_hbm.at[pl.ds(8, H - 8), pl.ds(b0, L)],
            idx_v.at[pl.ds(8, H - 8)],
        )

        def grp(g, carry):
            h0 = g * NBUF
            for b in range(NBUF):
                wait_gather(b, h0 + b)
                start_out(b, h0 + b)
            for b in range(NBUF):
                wait_out(b, h0 + b)
                start_gather(b, h0 + b + NBUF)
            return carry

        lax.fori_loop(0, ngrp - 1, grp, 0)

        h0 = (ngrp - 1) * NBUF
        for b in range(NBUF):
            wait_gather(b, h0 + b)
            start_out(b, h0 + b)
        for b in range(tail):
            wait_out(b, h0 + b)
            start_gather(b, ngrp * NBUF + b)
        for b in range(tail, NBUF):
            wait_out(b, h0 + b)
        for b in range(tail):
            wait_gather(b, ngrp * NBUF + b)
            start_out(b, ngrp * NBUF + b)
        for b in range(tail):
            wait_out(b, ngrp * NBUF + b)

    out = emb(Xt, W)
    return jnp.transpose(out, (1, 0, 2))


# speedup vs baseline: 1.0325x; 1.0325x over previous
import functools

import jax
import jax.numpy as jnp
from jax import lax
from jax.experimental import pallas as pl
from jax.experimental.pallas import tpu as pltpu
from jax.experimental.pallas import tpu_sc as plsc

NC = 2
NS = 16
NW = NC * NS
L = 128
NBUF = 7


def kernel(X, W):
    B, H = X.shape
    V, D = W.shape
    bpw = B // NW
    assert bpw * NW == B and bpw == L
    ngrp = H // NBUF
    tail = H - ngrp * NBUF

    Xt = X.T.astype(jnp.int32)

    mesh = plsc.VectorSubcoreMesh(core_axis_name="c", subcore_axis_name="s")

    @functools.partial(
        pl.kernel,
        out_type=jax.ShapeDtypeStruct((H, B, D), jnp.float32),
        mesh=mesh,
        scratch_types=[
            pltpu.VMEM((H, L), jnp.int32),
            [pltpu.VMEM((L, D), jnp.float32) for _ in range(NBUF)],
            [pltpu.SemaphoreType.DMA for _ in range(NBUF)],
            [pltpu.SemaphoreType.DMA for _ in range(NBUF)],
        ],
    )
    def emb(x_hbm, w_hbm, out_hbm, idx_v, bufs, gsems, osems):
        wid = lax.axis_index("s") * NC + lax.axis_index("c")
        b0 = wid * L
        pltpu.sync_copy(x_hbm.at[:, pl.ds(b0, L)], idx_v)

        def start_gather(b, h):
            pltpu.make_async_copy(
                w_hbm.at[idx_v.at[h]], bufs[b], gsems[b]
            ).start()

        def wait_gather(b, h):
            pltpu.make_async_copy(
                w_hbm.at[idx_v.at[h]], bufs[b], gsems[b]
            ).wait()

        def start_out(b, h):
            pltpu.make_async_copy(
                bufs[b], out_hbm.at[h, pl.ds(b0, L)], osems[b]
            ).start()

        def wait_out(b, h):
            pltpu.make_async_copy(
                bufs[b], out_hbm.at[h, pl.ds(b0, L)], osems[b]
            ).wait()

        LA = 3

        for k in range(LA):
            start_gather(k % NBUF, k)

        def step(b, c):
            wait_gather(b, c)
            start_out(b, c)
            b2 = (b + LA) % NBUF
            wait_out(b2, c - (NBUF - LA))
            start_gather(b2, c + LA)

        for c in range(NBUF):
            b = c % NBUF
            wait_gather(b, c)
            start_out(b, c)
            b2 = (c + LA) % NBUF
            if c + LA >= NBUF:
                wait_out(b2, c - (NBUF - LA))
            start_gather(b2, c + LA)

        def grp(g, carry):
            h0 = g * NBUF
            for b in range(NBUF):
                step(b, h0 + b)
            return carry

        nlast = H - NBUF - LA
        ngrid = nlast // NBUF
        lax.fori_loop(1, 1 + ngrid, grp, 0)

        for c in range(NBUF + ngrid * NBUF, H):
            b = c % NBUF
            wait_gather(b, c)
            start_out(b, c)
            b2 = (c + LA) % NBUF
            wait_out(b2, c - (NBUF - LA))
            if c + LA < H:
                start_gather(b2, c + LA)
        for c in range(H - (NBUF - LA), H):
            wait_out(c % NBUF, c)

    out = emb(Xt, W)
    return jnp.transpose(out, (1, 0, 2))


# speedup vs baseline: 1.0370x; 1.0044x over previous
import functools

import jax
import jax.numpy as jnp
from jax import lax
from jax.experimental import pallas as pl
from jax.experimental.pallas import tpu as pltpu
from jax.experimental.pallas import tpu_sc as plsc

NC = 2
NS = 16
NW = NC * NS
L = 128
NBUF = 7


def kernel(X, W):
    B, H = X.shape
    V, D = W.shape
    bpw = B // NW
    assert bpw * NW == B and bpw == L
    ngrp = H // NBUF
    tail = H - ngrp * NBUF

    Xt = X.T.astype(jnp.int32)

    mesh = plsc.VectorSubcoreMesh(core_axis_name="c", subcore_axis_name="s")

    @functools.partial(
        pl.kernel,
        out_type=jax.ShapeDtypeStruct((H, B, D), jnp.float32),
        mesh=mesh,
        scratch_types=[
            pltpu.VMEM((H, L), jnp.int32),
            [pltpu.VMEM((L, D), jnp.float32) for _ in range(NBUF)],
            [pltpu.SemaphoreType.DMA for _ in range(NBUF)],
            [pltpu.SemaphoreType.DMA for _ in range(NBUF)],
        ],
    )
    def emb(x_hbm, w_hbm, out_hbm, idx_v, bufs, gsems, osems):
        wid = lax.axis_index("s") * NC + lax.axis_index("c")
        b0 = wid * L
        pltpu.sync_copy(x_hbm.at[:, pl.ds(b0, L)], idx_v)

        def start_gather(b, h):
            pltpu.make_async_copy(
                w_hbm.at[idx_v.at[h]], bufs[b], gsems[b]
            ).start()

        def wait_gather(b, h):
            pltpu.make_async_copy(
                w_hbm.at[idx_v.at[h]], bufs[b], gsems[b]
            ).wait()

        def start_out(b, h):
            pltpu.make_async_copy(
                bufs[b], out_hbm.at[h, pl.ds(b0, L)], osems[b]
            ).start()

        def wait_out(b, h):
            pltpu.make_async_copy(
                bufs[b], out_hbm.at[h, pl.ds(b0, L)], osems[b]
            ).wait()

        LA = 4

        for k in range(LA):
            start_gather(k % NBUF, k)

        def step(b, c):
            wait_gather(b, c)
            start_out(b, c)
            b2 = (b + LA) % NBUF
            wait_out(b2, c - (NBUF - LA))
            start_gather(b2, c + LA)

        for c in range(NBUF):
            b = c % NBUF
            wait_gather(b, c)
            start_out(b, c)
            b2 = (c + LA) % NBUF
            if c + LA >= NBUF:
                wait_out(b2, c - (NBUF - LA))
            start_gather(b2, c + LA)

        def grp(g, carry):
            h0 = g * NBUF
            for b in range(NBUF):
                step(b, h0 + b)
            return carry

        nlast = H - NBUF - LA
        ngrid = nlast // NBUF
        lax.fori_loop(1, 1 + ngrid, grp, 0)

        for c in range(NBUF + ngrid * NBUF, H):
            b = c % NBUF
            wait_gather(b, c)
            start_out(b, c)
            b2 = (c + LA) % NBUF
            wait_out(b2, c - (NBUF - LA))
            if c + LA < H:
                start_gather(b2, c + LA)
        for c in range(H - (NBUF - LA), H):
            wait_out(c % NBUF, c)

    out = emb(Xt, W)
    return jnp.transpose(out, (1, 0, 2))


# speedup vs baseline: 1.0412x; 1.0041x over previous
import functools

import jax
import jax.numpy as jnp
from jax import lax
from jax.experimental import pallas as pl
from jax.experimental.pallas import tpu as pltpu
from jax.experimental.pallas import tpu_sc as plsc

NC = 2
NS = 16
NW = NC * NS
L = 128
NBUF = 7


def kernel(X, W):
    B, H = X.shape
    V, D = W.shape
    bpw = B // NW
    assert bpw * NW == B and bpw == L
    ngrp = H // NBUF
    tail = H - ngrp * NBUF

    Xt = X.T.astype(jnp.int32)

    mesh = plsc.VectorSubcoreMesh(core_axis_name="c", subcore_axis_name="s")

    @functools.partial(
        pl.kernel,
        out_type=jax.ShapeDtypeStruct((H, B, D), jnp.float32),
        mesh=mesh,
        scratch_types=[
            pltpu.VMEM((H, L), jnp.int32),
            [pltpu.VMEM((L, D), jnp.float32) for _ in range(NBUF)],
            [pltpu.SemaphoreType.DMA for _ in range(NBUF)],
            [pltpu.SemaphoreType.DMA for _ in range(NBUF)],
        ],
    )
    def emb(x_hbm, w_hbm, out_hbm, idx_v, bufs, gsems, osems):
        wid = lax.axis_index("s") * NC + lax.axis_index("c")
        b0 = wid * L
        pltpu.sync_copy(x_hbm.at[:, pl.ds(b0, L)], idx_v)

        def start_gather(b, h):
            pltpu.make_async_copy(
                w_hbm.at[idx_v.at[h]], bufs[b], gsems[b]
            ).start()

        def wait_gather(b, h):
            pltpu.make_async_copy(
                w_hbm.at[idx_v.at[h]], bufs[b], gsems[b]
            ).wait()

        def start_out(b, h):
            pltpu.make_async_copy(
                bufs[b], out_hbm.at[h, pl.ds(b0, L)], osems[b]
            ).start()

        def wait_out(b, h):
            pltpu.make_async_copy(
                bufs[b], out_hbm.at[h, pl.ds(b0, L)], osems[b]
            ).wait()

        LA = 5

        for k in range(LA):
            start_gather(k % NBUF, k)

        def step(b, c):
            wait_gather(b, c)
            start_out(b, c)
            b2 = (b + LA) % NBUF
            wait_out(b2, c - (NBUF - LA))
            start_gather(b2, c + LA)

        for c in range(NBUF):
            b = c % NBUF
            wait_gather(b, c)
            start_out(b, c)
            b2 = (c + LA) % NBUF
            if c + LA >= NBUF:
                wait_out(b2, c - (NBUF - LA))
            start_gather(b2, c + LA)

        def grp(g, carry):
            h0 = g * NBUF
            for b in range(NBUF):
                step(b, h0 + b)
            return carry

        nlast = H - NBUF - LA
        ngrid = nlast // NBUF
        lax.fori_loop(1, 1 + ngrid, grp, 0)

        for c in range(NBUF + ngrid * NBUF, H):
            b = c % NBUF
            wait_gather(b, c)
            start_out(b, c)
            b2 = (c + LA) % NBUF
            wait_out(b2, c - (NBUF - LA))
            if c + LA < H:
                start_gather(b2, c + LA)
        for c in range(H - (NBUF - LA), H):
            wait_out(c % NBUF, c)

    out = emb(Xt, W)
    return jnp.transpose(out, (1, 0, 2))


# speedup vs baseline: 1.0420x; 1.0007x over previous
import functools

import jax
import jax.numpy as jnp
from jax import lax
from jax.experimental import pallas as pl
from jax.experimental.pallas import tpu as pltpu
from jax.experimental.pallas import tpu_sc as plsc

NC = 2
NS = 16
NW = NC * NS
L = 128
NBUF = 7


def kernel(X, W):
    B, H = X.shape
    V, D = W.shape
    bpw = B // NW
    assert bpw * NW == B and bpw == L
    ngrp = H // NBUF
    tail = H - ngrp * NBUF

    Xt = X.T.astype(jnp.int32)

    mesh = plsc.VectorSubcoreMesh(core_axis_name="c", subcore_axis_name="s")

    @functools.partial(
        pl.kernel,
        out_type=jax.ShapeDtypeStruct((H, B, D), jnp.float32),
        mesh=mesh,
        scratch_types=[
            pltpu.VMEM((H, L), jnp.int32),
            [pltpu.VMEM((L, D), jnp.float32) for _ in range(NBUF)],
            [pltpu.SemaphoreType.DMA for _ in range(NBUF)],
            [pltpu.SemaphoreType.DMA for _ in range(NBUF)],
        ],
    )
    def emb(x_hbm, w_hbm, out_hbm, idx_v, bufs, gsems, osems):
        wid = lax.axis_index("s") * NC + lax.axis_index("c")
        b0 = wid * L
        pltpu.sync_copy(x_hbm.at[:, pl.ds(b0, L)], idx_v)

        def start_gather(b, h):
            pltpu.make_async_copy(
                w_hbm.at[idx_v.at[h]], bufs[b], gsems[b]
            ).start()

        def wait_gather(b, h):
            pltpu.make_async_copy(
                w_hbm.at[idx_v.at[h]], bufs[b], gsems[b]
            ).wait()

        def start_out(b, h):
            pltpu.make_async_copy(
                bufs[b], out_hbm.at[h, pl.ds(b0, L)], osems[b]
            ).start()

        def wait_out(b, h):
            pltpu.make_async_copy(
                bufs[b], out_hbm.at[h, pl.ds(b0, L)], osems[b]
            ).wait()

        LA = 6

        for k in range(LA):
            start_gather(k % NBUF, k)

        def step(b, c):
            wait_gather(b, c)
            start_out(b, c)
            b2 = (b + LA) % NBUF
            wait_out(b2, c - (NBUF - LA))
            start_gather(b2, c + LA)

        for c in range(NBUF):
            b = c % NBUF
            wait_gather(b, c)
            start_out(b, c)
            b2 = (c + LA) % NBUF
            if c + LA >= NBUF:
                wait_out(b2, c - (NBUF - LA))
            start_gather(b2, c + LA)

        def grp(g, carry):
            h0 = g * NBUF
            for b in range(NBUF):
                step(b, h0 + b)
            return carry

        nlast = H - NBUF - LA
        ngrid = nlast // NBUF
        lax.fori_loop(1, 1 + ngrid, grp, 0)

        for c in range(NBUF + ngrid * NBUF, H):
            b = c % NBUF
            wait_gather(b, c)
            start_out(b, c)
            b2 = (c + LA) % NBUF
            wait_out(b2, c - (NBUF - LA))
            if c + LA < H:
                start_gather(b2, c + LA)
        for c in range(H - (NBUF - LA), H):
            wait_out(c % NBUF, c)

    out = emb(Xt, W)
    return jnp.transpose(out, (1, 0, 2))
